# baseline (speedup 1.0000x reference)
import jax
import jax.numpy as jnp
from jax import lax
from jax.experimental import pallas as pl
from jax.experimental.pallas import tpu as pltpu
from jax.experimental.pallas import tpu_sc as plsc

NUM_CORES = 1
NUM_SUBCORES = 16
NUM_WORKERS = NUM_CORES * NUM_SUBCORES
LANES = 16
BATCH = 16384
PER_WORKER = BATCH // NUM_WORKERS


HALF = PER_WORKER // 2


def _sc_body(idx_hbm, loss_hbm, sigma_hbm, sl_hbm, sel_hbm,
             idx_v, loss_v, sel_v, sl_v, sem_g0, sem_g1, sem_l, sem_o):
    wid = lax.axis_index("s") * NUM_CORES + lax.axis_index("c")
    base = wid * PER_WORKER

    loss_cp = pltpu.async_copy(loss_hbm.at[pl.ds(base, PER_WORKER)], loss_v,
                               sem_l)
    pltpu.sync_copy(idx_hbm.at[pl.ds(base, PER_WORKER)], idx_v)
    g0 = pltpu.async_copy(sigma_hbm.at[idx_v.at[pl.ds(0, HALF)]],
                          sel_v.at[pl.ds(0, HALF)], sem_g0)
    g1 = pltpu.async_copy(sigma_hbm.at[idx_v.at[pl.ds(HALF, HALF)]],
                          sel_v.at[pl.ds(HALF, HALF)], sem_g1)
    loss_cp.wait()
    g0.wait()

    @pl.loop(0, HALF, step=LANES)
    def _(c0):
        sl_v[pl.ds(c0, LANES)] = sel_v[pl.ds(c0, LANES)] * loss_v[pl.ds(c0, LANES)]

    o0 = pltpu.async_copy(sl_v.at[pl.ds(0, HALF)],
                          sl_hbm.at[pl.ds(base, HALF)], sem_o)
    o1 = pltpu.async_copy(sel_v.at[pl.ds(0, HALF)],
                          sel_hbm.at[pl.ds(base, HALF)], sem_o)
    g1.wait()

    @pl.loop(HALF, PER_WORKER, step=LANES)
    def _(c0):
        sl_v[pl.ds(c0, LANES)] = sel_v[pl.ds(c0, LANES)] * loss_v[pl.ds(c0, LANES)]

    o2 = pltpu.async_copy(sl_v.at[pl.ds(HALF, HALF)],
                          sl_hbm.at[pl.ds(base + HALF, HALF)], sem_o)
    o3 = pltpu.async_copy(sel_v.at[pl.ds(HALF, HALF)],
                          sel_hbm.at[pl.ds(base + HALF, HALF)], sem_o)
    o0.wait()
    o1.wait()
    o2.wait()
    o3.wait()


def kernel(loss, basin_idx, sigma):
    idx = basin_idx.astype(jnp.int32)

    mesh = plsc.VectorSubcoreMesh(
        core_axis_name="c", subcore_axis_name="s",
        num_cores=NUM_CORES, num_subcores=NUM_SUBCORES,
    )
    out_type = (
        jax.ShapeDtypeStruct((BATCH,), jnp.float32),
        jax.ShapeDtypeStruct((BATCH,), jnp.float32),
    )
    scratch = [
        pltpu.VMEM((PER_WORKER,), jnp.int32),
        pltpu.VMEM((PER_WORKER,), jnp.float32),
        pltpu.VMEM((PER_WORKER,), jnp.float32),
        pltpu.VMEM((PER_WORKER,), jnp.float32),
        pltpu.SemaphoreType.DMA,
        pltpu.SemaphoreType.DMA,
        pltpu.SemaphoreType.DMA,
        pltpu.SemaphoreType.DMA,
    ]
    superloss, sel = pl.kernel(
        _sc_body, out_type=out_type, mesh=mesh, scratch_types=scratch,
    )(idx, loss, sigma)
    return superloss, sel


# speedup vs baseline: 1.0032x; 1.0032x over previous
import jax
import jax.numpy as jnp
from jax import lax
from jax.experimental import pallas as pl
from jax.experimental.pallas import tpu as pltpu
from jax.experimental.pallas import tpu_sc as plsc

NUM_CORES = 1
NUM_SUBCORES = 16
NUM_WORKERS = NUM_CORES * NUM_SUBCORES
LANES = 16
BATCH = 16384
PER_WORKER = BATCH // NUM_WORKERS


HALF = PER_WORKER // 2


def _sc_body(idx_hbm, loss_hbm, sigma_hbm, sl_hbm, sel_hbm,
             idx_v, loss_v, sel_v, sl_v, sem_g0, sem_g1, sem_l, sem_o):
    wid = lax.axis_index("s") * NUM_CORES + lax.axis_index("c")
    base = wid * PER_WORKER

    loss_cp = pltpu.async_copy(loss_hbm.at[pl.ds(base, PER_WORKER)], loss_v,
                               sem_l)
    pltpu.sync_copy(idx_hbm.at[pl.ds(base, PER_WORKER)], idx_v)
    g0 = pltpu.async_copy(sigma_hbm.at[idx_v.at[pl.ds(0, HALF)]],
                          sel_v.at[pl.ds(0, HALF)], sem_g0)
    g1 = pltpu.async_copy(sigma_hbm.at[idx_v.at[pl.ds(HALF, HALF)]],
                          sel_v.at[pl.ds(HALF, HALF)], sem_g1)
    loss_cp.wait()
    g0.wait()

    for c0 in range(0, HALF, LANES):
        sl_v[pl.ds(c0, LANES)] = sel_v[pl.ds(c0, LANES)] * loss_v[pl.ds(c0, LANES)]

    o0 = pltpu.async_copy(sl_v.at[pl.ds(0, HALF)],
                          sl_hbm.at[pl.ds(base, HALF)], sem_o)
    o1 = pltpu.async_copy(sel_v.at[pl.ds(0, HALF)],
                          sel_hbm.at[pl.ds(base, HALF)], sem_o)
    g1.wait()

    for c0 in range(HALF, PER_WORKER, LANES):
        sl_v[pl.ds(c0, LANES)] = sel_v[pl.ds(c0, LANES)] * loss_v[pl.ds(c0, LANES)]

    o2 = pltpu.async_copy(sl_v.at[pl.ds(HALF, HALF)],
                          sl_hbm.at[pl.ds(base + HALF, HALF)], sem_o)
    o3 = pltpu.async_copy(sel_v.at[pl.ds(HALF, HALF)],
                          sel_hbm.at[pl.ds(base + HALF, HALF)], sem_o)
    o0.wait()
    o1.wait()
    o2.wait()
    o3.wait()


def kernel(loss, basin_idx, sigma):
    idx = basin_idx.astype(jnp.int32)

    mesh = plsc.VectorSubcoreMesh(
        core_axis_name="c", subcore_axis_name="s",
        num_cores=NUM_CORES, num_subcores=NUM_SUBCORES,
    )
    out_type = (
        jax.ShapeDtypeStruct((BATCH,), jnp.float32),
        jax.ShapeDtypeStruct((BATCH,), jnp.float32),
    )
    scratch = [
        pltpu.VMEM((PER_WORKER,), jnp.int32),
        pltpu.VMEM((PER_WORKER,), jnp.float32),
        pltpu.VMEM((PER_WORKER,), jnp.float32),
        pltpu.VMEM((PER_WORKER,), jnp.float32),
        pltpu.SemaphoreType.DMA,
        pltpu.SemaphoreType.DMA,
        pltpu.SemaphoreType.DMA,
        pltpu.SemaphoreType.DMA,
    ]
    superloss, sel = pl.kernel(
        _sc_body, out_type=out_type, mesh=mesh, scratch_types=scratch,
    )(idx, loss, sigma)
    return superloss, sel


# speedup vs baseline: 1.0046x; 1.0014x over previous
import jax
import jax.numpy as jnp
from jax import lax
from jax.experimental import pallas as pl
from jax.experimental.pallas import tpu as pltpu
from jax.experimental.pallas import tpu_sc as plsc

NUM_CORES = 1
NUM_SUBCORES = 16
NUM_WORKERS = NUM_CORES * NUM_SUBCORES
LANES = 16
BATCH = 16384
PER_WORKER = BATCH // NUM_WORKERS


HALF = PER_WORKER // 2


def _sc_body(idx_hbm, loss_hbm, sigma_hbm, sl_hbm, sel_hbm,
             idx_v, loss_v, sel_v, sl_v, sem_g0, sem_g1, sem_l, sem_o):
    wid = lax.axis_index("s") * NUM_CORES + lax.axis_index("c")
    base = wid * PER_WORKER

    loss_cp = pltpu.async_copy(loss_hbm.at[pl.ds(base, PER_WORKER)], loss_v,
                               sem_l)
    pltpu.sync_copy(idx_hbm.at[pl.ds(base, PER_WORKER)], idx_v)
    g0 = pltpu.async_copy(sigma_hbm.at[idx_v.at[pl.ds(0, HALF)]],
                          sel_v.at[pl.ds(0, HALF)], sem_g0)
    g1 = pltpu.async_copy(sigma_hbm.at[idx_v.at[pl.ds(HALF, HALF)]],
                          sel_v.at[pl.ds(HALF, HALF)], sem_g1)
    loss_cp.wait()
    g0.wait()

    @pl.loop(0, HALF, step=LANES)
    def _(c0):
        sl_v[pl.ds(c0, LANES)] = sel_v[pl.ds(c0, LANES)] * loss_v[pl.ds(c0, LANES)]

    o0 = pltpu.async_copy(sl_v.at[pl.ds(0, HALF)],
                          sl_hbm.at[pl.ds(base, HALF)], sem_o)
    o1 = pltpu.async_copy(sel_v.at[pl.ds(0, HALF)],
                          sel_hbm.at[pl.ds(base, HALF)], sem_o)
    g1.wait()

    @pl.loop(HALF, PER_WORKER, step=LANES)
    def _(c0):
        sl_v[pl.ds(c0, LANES)] = sel_v[pl.ds(c0, LANES)] * loss_v[pl.ds(c0, LANES)]

    o2 = pltpu.async_copy(sl_v.at[pl.ds(HALF, HALF)],
                          sl_hbm.at[pl.ds(base + HALF, HALF)], sem_o)
    o3 = pltpu.async_copy(sel_v.at[pl.ds(HALF, HALF)],
                          sel_hbm.at[pl.ds(base + HALF, HALF)], sem_o)
    o0.wait()
    o1.wait()
    o2.wait()
    o3.wait()


def kernel(loss, basin_idx, sigma):
    idx = basin_idx.astype(jnp.int32)

    mesh = plsc.VectorSubcoreMesh(
        core_axis_name="c", subcore_axis_name="s",
        num_cores=NUM_CORES, num_subcores=NUM_SUBCORES,
    )
    out_type = (
        jax.ShapeDtypeStruct((BATCH,), jnp.float32),
        jax.ShapeDtypeStruct((BATCH,), jnp.float32),
    )
    scratch = [
        pltpu.VMEM((PER_WORKER,), jnp.int32),
        pltpu.VMEM((PER_WORKER,), jnp.float32),
        pltpu.VMEM((PER_WORKER,), jnp.float32),
        pltpu.VMEM((PER_WORKER,), jnp.float32),
        pltpu.SemaphoreType.DMA,
        pltpu.SemaphoreType.DMA,
        pltpu.SemaphoreType.DMA,
        pltpu.SemaphoreType.DMA,
    ]
    superloss, sel = pl.kernel(
        _sc_body, out_type=out_type, mesh=mesh, scratch_types=scratch,
    )(idx, loss, sigma)
    return superloss, sel
